# baseline (speedup 1.0000x reference)
import functools

import jax
import jax.numpy as jnp
from jax import lax
from jax.experimental import pallas as pl
from jax.experimental.pallas import tpu as pltpu
from jax.experimental.pallas import tpu_sc as plsc


NC = 2
NS = 16
L = 16
NW = NC * NS


def _sc_mesh():
  return plsc.VectorSubcoreMesh(core_axis_name="c", subcore_axis_name="s")


def _sc_degree_histogram(e_total, n_nodes):
  ec = 10240
  ec_last = e_total - (NW - 1) * ec
  assert ec_last > 0 and ec_last % 512 == 0
  npad = ((n_nodes + NS * L - 1) // (NS * L)) * (NS * L)
  rs = npad // NS

  @functools.partial(
      pl.kernel,
      out_type=jax.ShapeDtypeStruct((NC * npad,), jnp.float32),
      mesh=_sc_mesh(),
      compiler_params=pltpu.CompilerParams(needs_layout_passes=False),
      scratch_types=[
          pltpu.VMEM((2, ec), jnp.int32),
          pltpu.VMEM((npad,), jnp.float32),
          pltpu.VMEM_SHARED((NS, npad), jnp.float32),
          pltpu.VMEM((NS, rs), jnp.float32),
          pltpu.VMEM((rs,), jnp.float32),
          pltpu.SemaphoreType.DMA,
      ],
  )
  def hist_kernel(ei_hbm, deg_hbm, idx_v, hist_v, shared, red_v, acc_v, sem):
    cid = lax.axis_index("c")
    sid = lax.axis_index("s")
    wid = cid * NS + sid

    zeros = jnp.zeros((L,), jnp.float32)
    ones = jnp.ones((L,), jnp.float32)
    hu = 8

    def do_hist(csz):
      half = csz // 2
      assert half % (L * hu) == 0
      off = wid * ec
      cp0 = pltpu.async_copy(
          ei_hbm.at[:, pl.ds(off, half)], idx_v.at[:, pl.ds(0, half)], sem)
      cp1 = pltpu.async_copy(
          ei_hbm.at[:, pl.ds(off + half, half)],
          idx_v.at[:, pl.ds(half, half)], sem)

      zu = 8
      assert npad % (L * zu) == 0

      def zbody(i, _):
        for u in range(zu):
          hist_v[pl.ds((i * zu + u) * L, L)] = zeros
        return 0

      lax.fori_loop(0, npad // (L * zu), zbody, 0)

      def hbody(e, _):
        for u in range(hu):
          idx16 = idx_v[1, pl.ds((e * hu + u) * L, L)]
          plsc.addupdate_scatter(hist_v, [idx16], ones)
        return 0

      cp0.wait()
      lax.fori_loop(0, half // (L * hu), hbody, 0)
      cp1.wait()
      lax.fori_loop(half // (L * hu), csz // (L * hu), hbody, 0)

    @pl.when(wid < NW - 1)
    def _():
      do_hist(ec)

    @pl.when(wid == NW - 1)
    def _():
      do_hist(ec_last)

    pltpu.sync_copy(hist_v, shared.at[sid])
    plsc.subcore_barrier()

    pltpu.sync_copy(shared.at[:, pl.ds(sid * rs, rs)], red_v)

    def rbody(j, _):
      acc = red_v[0, pl.ds(j * L, L)]
      for k in range(1, NS):
        acc = acc + red_v[k, pl.ds(j * L, L)]
      acc_v[pl.ds(j * L, L)] = acc
      return 0

    lax.fori_loop(0, rs // L, rbody, 0)

    pltpu.sync_copy(acc_v, deg_hbm.at[pl.ds(cid * npad + sid * rs, rs)])

  return hist_kernel, npad


def _tc_matmul(x, W, b2, block_rows):
  n, dfeat = x.shape
  emb = W.shape[0]
  assert n % block_rows == 0

  def body(x_ref, w_ref, b_ref, o_ref):
    o_ref[...] = lax.dot_general(
        x_ref[...], w_ref[...],
        dimension_numbers=(((1,), (1,)), ((), ())),
        preferred_element_type=jnp.float32) + b_ref[...]

  return pl.pallas_call(
      body,
      grid=(n // block_rows,),
      in_specs=[
          pl.BlockSpec((block_rows, dfeat), lambda i: (i, 0)),
          pl.BlockSpec((emb, dfeat), lambda i: (0, 0)),
          pl.BlockSpec((1, emb), lambda i: (0, 0)),
      ],
      out_specs=pl.BlockSpec((block_rows, emb), lambda i: (i, 0)),
      out_shape=jax.ShapeDtypeStruct((n, emb), jnp.float32),
  )(x, W, b2)


def _sc_combine(deg2, degree_table, feat, n_nodes):
  npad = deg2.shape[0] // NC
  vocab, emb = degree_table.shape
  rs = npad // NW
  last = n_nodes - (NW - 1) * rs
  nch = 4
  gc = rs // nch
  assert rs % (nch * L) == 0 and gc <= 128 and last == gc and last % 8 == 0

  @functools.partial(
      pl.kernel,
      out_type=jax.ShapeDtypeStruct((n_nodes, emb), jnp.float32),
      mesh=_sc_mesh(),
      compiler_params=pltpu.CompilerParams(needs_layout_passes=False),
      scratch_types=[
          pltpu.VMEM((rs,), jnp.float32),
          pltpu.VMEM((rs,), jnp.float32),
          pltpu.VMEM((rs,), jnp.int32),
          pltpu.VMEM((rs, emb), jnp.float32),
          pltpu.VMEM_SHARED((vocab, emb), jnp.float32),
          pltpu.SemaphoreType.DMA,
          pltpu.SemaphoreType.DMA,
          pltpu.SemaphoreType.DMA,
          pltpu.SemaphoreType.DMA,
          pltpu.SemaphoreType.DMA,
          pltpu.SemaphoreType.DMA,
      ],
  )
  def combine_kernel(deg_hbm, tab_hbm, feat_hbm, out_hbm,
                     da_v, db_v, ii_v, acc_v, tab_sp, sem_d, sem_o,
                     sc0, sc1, sc2, sc3):
    scs = [sc0, sc1, sc2, sc3]
    cid = lax.axis_index("c")
    sid = lax.axis_index("s")
    wid = cid * NS + sid
    base = wid * rs
    is_last = wid == NW - 1

    cpa = pltpu.async_copy(deg_hbm.at[pl.ds(base, rs)], da_v, sem_d)
    cpb = pltpu.async_copy(deg_hbm.at[pl.ds(npad + base, rs)], db_v, sem_d)

    cp0 = pltpu.async_copy(
        feat_hbm.at[pl.ds(base, gc)], acc_v.at[pl.ds(0, gc)], scs[0])

    @pl.when(~is_last)
    def _():
      for k in range(1, nch):
        pltpu.async_copy(feat_hbm.at[pl.ds(base + k * gc, gc)],
                         acc_v.at[pl.ds(k * gc, gc)], scs[k])

    tslice = vocab // NS
    pltpu.sync_copy(tab_hbm.at[pl.ds(sid * tslice, tslice)],
                    tab_sp.at[pl.ds(sid * tslice, tslice)])

    cpa.wait()
    cpb.wait()

    def ibody(j, _):
      s = da_v[pl.ds(j * L, L)] + db_v[pl.ds(j * L, L)]
      ii_v[pl.ds(j * L, L)] = jnp.minimum(s.astype(jnp.int32), vocab - 1)
      return 0

    lax.fori_loop(0, rs // L, ibody, 0)

    plsc.subcore_barrier()

    def fire_gather(k):
      return pltpu.async_copy(tab_sp.at[ii_v.at[pl.ds(k * gc, gc)]],
                              acc_v.at[pl.ds(k * gc, gc)], scs[k], add=True)

    def fire_out(k):
      pltpu.async_copy(acc_v.at[pl.ds(k * gc, gc)],
                       out_hbm.at[pl.ds(base + k * gc, gc)], sem_o)

    cp0.wait()
    g0 = fire_gather(0)

    @pl.when(~is_last)
    def _():
      gprev = g0
      for k in range(1, nch):
        pltpu.make_async_copy(feat_hbm.at[pl.ds(base + k * gc, gc)],
                              acc_v.at[pl.ds(k * gc, gc)], scs[k]).wait()
        gk = fire_gather(k)
        gprev.wait()
        fire_out(k - 1)
        gprev = gk
      gprev.wait()
      fire_out(nch - 1)
      for k in range(nch):
        pltpu.make_async_copy(acc_v.at[pl.ds(k * gc, gc)],
                              out_hbm.at[pl.ds(base + k * gc, gc)],
                              sem_o).wait()

    @pl.when(is_last)
    def _():
      g0.wait()
      fire_out(0)
      pltpu.make_async_copy(acc_v.at[pl.ds(0, gc)],
                            out_hbm.at[pl.ds(base, gc)], sem_o).wait()

  return combine_kernel(deg2, degree_table, feat)


def kernel(x, edge_index, W, b, degree_table):
  n = x.shape[0]
  e_total = edge_index.shape[1]
  hist_kernel, _ = _sc_degree_histogram(e_total, n)
  deg2 = hist_kernel(edge_index)
  feat = _tc_matmul(x, W, b.reshape(1, -1), block_rows=2000)
  return _sc_combine(deg2, degree_table, feat, n)


# speedup vs baseline: 1.0163x; 1.0163x over previous
import functools

import jax
import jax.numpy as jnp
from jax import lax
from jax.experimental import pallas as pl
from jax.experimental.pallas import tpu as pltpu
from jax.experimental.pallas import tpu_sc as plsc


NC = 2
NS = 16
L = 16
NW = NC * NS


def _sc_mesh():
  return plsc.VectorSubcoreMesh(core_axis_name="c", subcore_axis_name="s")


def _sc_degree_histogram(e_total, n_nodes):
  ec = 10240
  ec_last = e_total - (NW - 1) * ec
  assert ec_last > 0 and ec_last % 512 == 0
  npad = ((n_nodes + NS * L - 1) // (NS * L)) * (NS * L)
  rs = npad // NS

  @functools.partial(
      pl.kernel,
      out_type=jax.ShapeDtypeStruct((NC * npad,), jnp.int32),
      mesh=_sc_mesh(),
      compiler_params=pltpu.CompilerParams(needs_layout_passes=False),
      scratch_types=[
          pltpu.VMEM((2, ec), jnp.int32),
          pltpu.VMEM((npad,), jnp.int32),
          pltpu.VMEM_SHARED((NS, npad), jnp.int32),
          pltpu.VMEM((NS, rs), jnp.int32),
          pltpu.VMEM((rs,), jnp.int32),
          pltpu.SemaphoreType.DMA,
      ],
  )
  def hist_kernel(ei_hbm, deg_hbm, idx_v, hist_v, shared, red_v, acc_v, sem):
    cid = lax.axis_index("c")
    sid = lax.axis_index("s")
    wid = cid * NS + sid

    zeros = jnp.zeros((L,), jnp.int32)
    ones = jnp.ones((L,), jnp.int32)
    hu = 8

    def do_hist(csz):
      half = csz // 2
      assert half % (L * hu) == 0
      off = wid * ec
      cp0 = pltpu.async_copy(
          ei_hbm.at[:, pl.ds(off, half)], idx_v.at[:, pl.ds(0, half)], sem)
      cp1 = pltpu.async_copy(
          ei_hbm.at[:, pl.ds(off + half, half)],
          idx_v.at[:, pl.ds(half, half)], sem)

      zu = 8
      assert npad % (L * zu) == 0

      def zbody(i, _):
        for u in range(zu):
          hist_v[pl.ds((i * zu + u) * L, L)] = zeros
        return 0

      lax.fori_loop(0, npad // (L * zu), zbody, 0)

      def hbody(e, _):
        for u in range(hu):
          idx16 = idx_v[1, pl.ds((e * hu + u) * L, L)]
          plsc.addupdate_scatter(hist_v, [idx16], ones)
        return 0

      cp0.wait()
      lax.fori_loop(0, half // (L * hu), hbody, 0)
      cp1.wait()
      lax.fori_loop(half // (L * hu), csz // (L * hu), hbody, 0)

    @pl.when(wid < NW - 1)
    def _():
      do_hist(ec)

    @pl.when(wid == NW - 1)
    def _():
      do_hist(ec_last)

    pltpu.sync_copy(hist_v, shared.at[sid])
    plsc.subcore_barrier()

    pltpu.sync_copy(shared.at[:, pl.ds(sid * rs, rs)], red_v)

    def rbody(j, _):
      acc = red_v[0, pl.ds(j * L, L)]
      for k in range(1, NS):
        acc = acc + red_v[k, pl.ds(j * L, L)]
      acc_v[pl.ds(j * L, L)] = acc
      return 0

    lax.fori_loop(0, rs // L, rbody, 0)

    pltpu.sync_copy(acc_v, deg_hbm.at[pl.ds(cid * npad + sid * rs, rs)])

  return hist_kernel, npad


def _tc_matmul(x, W, b2, block_rows):
  n, dfeat = x.shape
  emb = W.shape[0]
  assert n % block_rows == 0

  def body(x_ref, w_ref, b_ref, o_ref):
    o_ref[...] = lax.dot_general(
        x_ref[...], w_ref[...],
        dimension_numbers=(((1,), (1,)), ((), ())),
        preferred_element_type=jnp.float32) + b_ref[...]

  return pl.pallas_call(
      body,
      grid=(n // block_rows,),
      in_specs=[
          pl.BlockSpec((block_rows, dfeat), lambda i: (i, 0)),
          pl.BlockSpec((emb, dfeat), lambda i: (0, 0)),
          pl.BlockSpec((1, emb), lambda i: (0, 0)),
      ],
      out_specs=pl.BlockSpec((block_rows, emb), lambda i: (i, 0)),
      out_shape=jax.ShapeDtypeStruct((n, emb), jnp.float32),
  )(x, W, b2)


def _sc_combine(deg2, degree_table, feat, n_nodes):
  npad = deg2.shape[0] // NC
  vocab, emb = degree_table.shape
  rs = npad // NW
  last = n_nodes - (NW - 1) * rs
  nch = 4
  gc = rs // nch
  assert rs % (nch * L) == 0 and gc <= 128 and last == gc and last % 8 == 0

  @functools.partial(
      pl.kernel,
      out_type=jax.ShapeDtypeStruct((n_nodes, emb), jnp.float32),
      mesh=_sc_mesh(),
      compiler_params=pltpu.CompilerParams(needs_layout_passes=False),
      scratch_types=[
          pltpu.VMEM((rs,), jnp.int32),
          pltpu.VMEM((rs,), jnp.int32),
          pltpu.VMEM((rs,), jnp.int32),
          pltpu.VMEM((rs, emb), jnp.float32),
          pltpu.VMEM_SHARED((vocab, emb), jnp.float32),
          pltpu.SemaphoreType.DMA,
          pltpu.SemaphoreType.DMA,
          pltpu.SemaphoreType.DMA,
          pltpu.SemaphoreType.DMA,
          pltpu.SemaphoreType.DMA,
          pltpu.SemaphoreType.DMA,
      ],
  )
  def combine_kernel(deg_hbm, tab_hbm, feat_hbm, out_hbm,
                     da_v, db_v, ii_v, acc_v, tab_sp, sem_d, sem_o,
                     sc0, sc1, sc2, sc3):
    scs = [sc0, sc1, sc2, sc3]
    cid = lax.axis_index("c")
    sid = lax.axis_index("s")
    wid = cid * NS + sid
    base = wid * rs
    is_last = wid == NW - 1

    cpa = pltpu.async_copy(deg_hbm.at[pl.ds(base, rs)], da_v, sem_d)
    cpb = pltpu.async_copy(deg_hbm.at[pl.ds(npad + base, rs)], db_v, sem_d)

    cp0 = pltpu.async_copy(
        feat_hbm.at[pl.ds(base, gc)], acc_v.at[pl.ds(0, gc)], scs[0])

    @pl.when(~is_last)
    def _():
      for k in range(1, nch):
        pltpu.async_copy(feat_hbm.at[pl.ds(base + k * gc, gc)],
                         acc_v.at[pl.ds(k * gc, gc)], scs[k])

    tslice = vocab // NS
    pltpu.sync_copy(tab_hbm.at[pl.ds(sid * tslice, tslice)],
                    tab_sp.at[pl.ds(sid * tslice, tslice)])

    cpa.wait()
    cpb.wait()

    def ibody(j, _):
      s = da_v[pl.ds(j * L, L)] + db_v[pl.ds(j * L, L)]
      ii_v[pl.ds(j * L, L)] = jnp.minimum(s, vocab - 1)
      return 0

    lax.fori_loop(0, rs // L, ibody, 0)

    plsc.subcore_barrier()

    def fire_gather(k):
      return pltpu.async_copy(tab_sp.at[ii_v.at[pl.ds(k * gc, gc)]],
                              acc_v.at[pl.ds(k * gc, gc)], scs[k], add=True)

    def fire_out(k):
      pltpu.async_copy(acc_v.at[pl.ds(k * gc, gc)],
                       out_hbm.at[pl.ds(base + k * gc, gc)], sem_o)

    cp0.wait()
    g0 = fire_gather(0)

    @pl.when(~is_last)
    def _():
      gprev = g0
      for k in range(1, nch):
        pltpu.make_async_copy(feat_hbm.at[pl.ds(base + k * gc, gc)],
                              acc_v.at[pl.ds(k * gc, gc)], scs[k]).wait()
        gk = fire_gather(k)
        gprev.wait()
        fire_out(k - 1)
        gprev = gk
      gprev.wait()
      fire_out(nch - 1)
      for k in range(nch):
        pltpu.make_async_copy(acc_v.at[pl.ds(k * gc, gc)],
                              out_hbm.at[pl.ds(base + k * gc, gc)],
                              sem_o).wait()

    @pl.when(is_last)
    def _():
      g0.wait()
      fire_out(0)
      pltpu.make_async_copy(acc_v.at[pl.ds(0, gc)],
                            out_hbm.at[pl.ds(base, gc)], sem_o).wait()

  return combine_kernel(deg2, degree_table, feat)


def kernel(x, edge_index, W, b, degree_table):
  n = x.shape[0]
  e_total = edge_index.shape[1]
  hist_kernel, _ = _sc_degree_histogram(e_total, n)
  deg2 = hist_kernel(edge_index)
  feat = _tc_matmul(x, W, b.reshape(1, -1), block_rows=2000)
  return _sc_combine(deg2, degree_table, feat, n)


# speedup vs baseline: 1.0874x; 1.0699x over previous
import functools

import jax
import jax.numpy as jnp
from jax import lax
from jax.experimental import pallas as pl
from jax.experimental.pallas import tpu as pltpu
from jax.experimental.pallas import tpu_sc as plsc


NC = 2
NS = 16
L = 16
NW = NC * NS


def _sc_mesh():
  return plsc.VectorSubcoreMesh(core_axis_name="c", subcore_axis_name="s")


def _sc_degree_histogram(e_total, n_nodes):
  ec = 10240
  ec_last = e_total - (NW - 1) * ec
  assert ec_last > 0 and ec_last % 512 == 0
  npad = ((n_nodes + NS * L - 1) // (NS * L)) * (NS * L)
  rs = npad // NS

  @functools.partial(
      pl.kernel,
      out_type=jax.ShapeDtypeStruct((NC * npad,), jnp.int32),
      mesh=_sc_mesh(),
      compiler_params=pltpu.CompilerParams(needs_layout_passes=False),
      scratch_types=[
          pltpu.VMEM((2, ec), jnp.int32),
          pltpu.VMEM((npad,), jnp.int32),
          pltpu.VMEM_SHARED((NS, npad), jnp.int32),
          pltpu.VMEM((NS, rs), jnp.int32),
          pltpu.VMEM((rs,), jnp.int32),
          pltpu.SemaphoreType.DMA,
      ],
  )
  def hist_kernel(ei_hbm, deg_hbm, idx_v, hist_v, shared, red_v, acc_v, sem):
    cid = lax.axis_index("c")
    sid = lax.axis_index("s")
    wid = cid * NS + sid

    zeros = jnp.zeros((L,), jnp.int32)
    ones = jnp.ones((L,), jnp.int32)
    hu = 16

    def do_hist(csz):
      half = csz // 2
      assert half % (L * hu) == 0
      off = wid * ec
      cp0 = pltpu.async_copy(
          ei_hbm.at[:, pl.ds(off, half)], idx_v.at[:, pl.ds(0, half)], sem)
      cp1 = pltpu.async_copy(
          ei_hbm.at[:, pl.ds(off + half, half)],
          idx_v.at[:, pl.ds(half, half)], sem)

      zu = 8
      assert npad % (L * zu) == 0

      def zbody(i, _):
        for u in range(zu):
          hist_v[pl.ds((i * zu + u) * L, L)] = zeros
        return 0

      lax.fori_loop(0, npad // (L * zu), zbody, 0)

      def hbody(e, _):
        idxs = [idx_v[1, pl.ds((e * hu + u) * L, L)] for u in range(hu)]
        for u in range(hu):
          plsc.addupdate_scatter(hist_v, [idxs[u]], ones)
        return 0

      cp0.wait()
      lax.fori_loop(0, half // (L * hu), hbody, 0)
      cp1.wait()
      lax.fori_loop(half // (L * hu), csz // (L * hu), hbody, 0)

    @pl.when(wid < NW - 1)
    def _():
      do_hist(ec)

    @pl.when(wid == NW - 1)
    def _():
      do_hist(ec_last)

    pltpu.sync_copy(hist_v, shared.at[sid])
    plsc.subcore_barrier()

    pltpu.sync_copy(shared.at[:, pl.ds(sid * rs, rs)], red_v)

    def rbody(j, _):
      acc = red_v[0, pl.ds(j * L, L)]
      for k in range(1, NS):
        acc = acc + red_v[k, pl.ds(j * L, L)]
      acc_v[pl.ds(j * L, L)] = acc
      return 0

    lax.fori_loop(0, rs // L, rbody, 0)

    pltpu.sync_copy(acc_v, deg_hbm.at[pl.ds(cid * npad + sid * rs, rs)])

  return hist_kernel, npad


def _tc_matmul(x, W, b2, block_rows):
  n, dfeat = x.shape
  emb = W.shape[0]
  assert n % block_rows == 0

  def body(x_ref, w_ref, b_ref, o_ref):
    o_ref[...] = lax.dot_general(
        x_ref[...], w_ref[...],
        dimension_numbers=(((1,), (1,)), ((), ())),
        preferred_element_type=jnp.float32) + b_ref[...]

  return pl.pallas_call(
      body,
      grid=(n // block_rows,),
      in_specs=[
          pl.BlockSpec((block_rows, dfeat), lambda i: (i, 0)),
          pl.BlockSpec((emb, dfeat), lambda i: (0, 0)),
          pl.BlockSpec((1, emb), lambda i: (0, 0)),
      ],
      out_specs=pl.BlockSpec((block_rows, emb), lambda i: (i, 0)),
      out_shape=jax.ShapeDtypeStruct((n, emb), jnp.float32),
  )(x, W, b2)


def _sc_combine(deg2, degree_table, feat, n_nodes):
  npad = deg2.shape[0] // NC
  vocab, emb = degree_table.shape
  rs = npad // NW
  last = n_nodes - (NW - 1) * rs
  nch = 4
  gc = rs // nch
  assert rs % (nch * L) == 0 and gc <= 128 and last == gc and last % 8 == 0

  @functools.partial(
      pl.kernel,
      out_type=jax.ShapeDtypeStruct((n_nodes, emb), jnp.float32),
      mesh=_sc_mesh(),
      compiler_params=pltpu.CompilerParams(needs_layout_passes=False),
      scratch_types=[
          pltpu.VMEM((rs,), jnp.int32),
          pltpu.VMEM((rs,), jnp.int32),
          pltpu.VMEM((rs,), jnp.int32),
          pltpu.VMEM((rs, emb), jnp.float32),
          pltpu.VMEM_SHARED((vocab, emb), jnp.float32),
          pltpu.SemaphoreType.DMA,
          pltpu.SemaphoreType.DMA,
          pltpu.SemaphoreType.DMA,
          pltpu.SemaphoreType.DMA,
          pltpu.SemaphoreType.DMA,
          pltpu.SemaphoreType.DMA,
      ],
  )
  def combine_kernel(deg_hbm, tab_hbm, feat_hbm, out_hbm,
                     da_v, db_v, ii_v, acc_v, tab_sp, sem_d, sem_o,
                     sc0, sc1, sc2, sc3):
    scs = [sc0, sc1, sc2, sc3]
    cid = lax.axis_index("c")
    sid = lax.axis_index("s")
    wid = cid * NS + sid
    base = wid * rs
    is_last = wid == NW - 1

    cpa = pltpu.async_copy(deg_hbm.at[pl.ds(base, rs)], da_v, sem_d)
    cpb = pltpu.async_copy(deg_hbm.at[pl.ds(npad + base, rs)], db_v, sem_d)

    cp0 = pltpu.async_copy(
        feat_hbm.at[pl.ds(base, gc)], acc_v.at[pl.ds(0, gc)], scs[0])

    @pl.when(~is_last)
    def _():
      for k in range(1, nch):
        pltpu.async_copy(feat_hbm.at[pl.ds(base + k * gc, gc)],
                         acc_v.at[pl.ds(k * gc, gc)], scs[k])

    tslice = vocab // NS
    pltpu.sync_copy(tab_hbm.at[pl.ds(sid * tslice, tslice)],
                    tab_sp.at[pl.ds(sid * tslice, tslice)])

    cpa.wait()
    cpb.wait()

    def ibody(j, _):
      s = da_v[pl.ds(j * L, L)] + db_v[pl.ds(j * L, L)]
      ii_v[pl.ds(j * L, L)] = jnp.minimum(s, vocab - 1)
      return 0

    lax.fori_loop(0, rs // L, ibody, 0)

    plsc.subcore_barrier()

    def fire_gather(k):
      return pltpu.async_copy(tab_sp.at[ii_v.at[pl.ds(k * gc, gc)]],
                              acc_v.at[pl.ds(k * gc, gc)], scs[k], add=True)

    def fire_out(k):
      pltpu.async_copy(acc_v.at[pl.ds(k * gc, gc)],
                       out_hbm.at[pl.ds(base + k * gc, gc)], sem_o)

    cp0.wait()
    g0 = fire_gather(0)

    @pl.when(~is_last)
    def _():
      gprev = g0
      for k in range(1, nch):
        pltpu.make_async_copy(feat_hbm.at[pl.ds(base + k * gc, gc)],
                              acc_v.at[pl.ds(k * gc, gc)], scs[k]).wait()
        gk = fire_gather(k)
        gprev.wait()
        fire_out(k - 1)
        gprev = gk
      gprev.wait()
      fire_out(nch - 1)
      for k in range(nch):
        pltpu.make_async_copy(acc_v.at[pl.ds(k * gc, gc)],
                              out_hbm.at[pl.ds(base + k * gc, gc)],
                              sem_o).wait()

    @pl.when(is_last)
    def _():
      g0.wait()
      fire_out(0)
      pltpu.make_async_copy(acc_v.at[pl.ds(0, gc)],
                            out_hbm.at[pl.ds(base, gc)], sem_o).wait()

  return combine_kernel(deg2, degree_table, feat)


def kernel(x, edge_index, W, b, degree_table):
  n = x.shape[0]
  e_total = edge_index.shape[1]
  hist_kernel, _ = _sc_degree_histogram(e_total, n)
  deg2 = hist_kernel(edge_index)
  feat = _tc_matmul(x, W, b.reshape(1, -1), block_rows=2000)
  return _sc_combine(deg2, degree_table, feat, n)


# speedup vs baseline: 1.0901x; 1.0025x over previous
import functools

import jax
import jax.numpy as jnp
from jax import lax
from jax.experimental import pallas as pl
from jax.experimental.pallas import tpu as pltpu
from jax.experimental.pallas import tpu_sc as plsc


NC = 2
NS = 16
L = 16
NW = NC * NS


def _sc_mesh():
  return plsc.VectorSubcoreMesh(core_axis_name="c", subcore_axis_name="s")


def _sc_degree_histogram(e_total, n_nodes):
  ec = 10240
  ec_last = e_total - (NW - 1) * ec
  assert ec_last > 0 and ec_last % 512 == 0
  npad = ((n_nodes + NS * L - 1) // (NS * L)) * (NS * L)
  rs = npad // NS

  @functools.partial(
      pl.kernel,
      out_type=jax.ShapeDtypeStruct((NC * npad,), jnp.int32),
      mesh=_sc_mesh(),
      compiler_params=pltpu.CompilerParams(needs_layout_passes=False),
      scratch_types=[
          pltpu.VMEM((2, ec), jnp.int32),
          pltpu.VMEM((npad,), jnp.int32),
          pltpu.VMEM_SHARED((NS, npad), jnp.int32),
          pltpu.VMEM((NS, rs), jnp.int32),
          pltpu.VMEM((rs,), jnp.int32),
          pltpu.SemaphoreType.DMA,
      ],
  )
  def hist_kernel(ei_hbm, deg_hbm, idx_v, hist_v, shared, red_v, acc_v, sem):
    cid = lax.axis_index("c")
    sid = lax.axis_index("s")
    wid = cid * NS + sid

    zeros = jnp.zeros((L,), jnp.int32)
    ones = jnp.ones((L,), jnp.int32)
    hu = 16

    def do_hist(csz):
      half = csz // 2
      assert half % (L * hu) == 0
      off = wid * ec
      cp0 = pltpu.async_copy(
          ei_hbm.at[:, pl.ds(off, half)], idx_v.at[:, pl.ds(0, half)], sem)
      cp1 = pltpu.async_copy(
          ei_hbm.at[:, pl.ds(off + half, half)],
          idx_v.at[:, pl.ds(half, half)], sem)

      zu = 8
      assert npad % (L * zu) == 0

      def zbody(i, _):
        for u in range(zu):
          hist_v[pl.ds((i * zu + u) * L, L)] = zeros
        return 0

      lax.fori_loop(0, npad // (L * zu), zbody, 0)

      def hbody(e, _):
        idxs = [idx_v[1, pl.ds((e * hu + u) * L, L)] for u in range(hu)]
        for u in range(hu):
          plsc.addupdate_scatter(hist_v, [idxs[u]], ones)
        return 0

      cp0.wait()
      lax.fori_loop(0, half // (L * hu), hbody, 0)
      cp1.wait()
      lax.fori_loop(half // (L * hu), csz // (L * hu), hbody, 0)

    @pl.when(wid < NW - 1)
    def _():
      do_hist(ec)

    @pl.when(wid == NW - 1)
    def _():
      do_hist(ec_last)

    pltpu.sync_copy(hist_v, shared.at[sid])
    plsc.subcore_barrier()

    pltpu.sync_copy(shared.at[:, pl.ds(sid * rs, rs)], red_v)

    def rbody(j, _):
      vals = [red_v[k, pl.ds(j * L, L)] for k in range(NS)]
      while len(vals) > 1:
        vals = [a + b for a, b in zip(vals[::2], vals[1::2])]
      acc_v[pl.ds(j * L, L)] = vals[0]
      return 0

    lax.fori_loop(0, rs // L, rbody, 0)

    pltpu.sync_copy(acc_v, deg_hbm.at[pl.ds(cid * npad + sid * rs, rs)])

  return hist_kernel, npad


def _tc_matmul(x, W, b2, block_rows):
  n, dfeat = x.shape
  emb = W.shape[0]
  assert n % block_rows == 0

  def body(x_ref, w_ref, b_ref, o_ref):
    o_ref[...] = lax.dot_general(
        x_ref[...], w_ref[...],
        dimension_numbers=(((1,), (1,)), ((), ())),
        preferred_element_type=jnp.float32) + b_ref[...]

  return pl.pallas_call(
      body,
      grid=(n // block_rows,),
      in_specs=[
          pl.BlockSpec((block_rows, dfeat), lambda i: (i, 0)),
          pl.BlockSpec((emb, dfeat), lambda i: (0, 0)),
          pl.BlockSpec((1, emb), lambda i: (0, 0)),
      ],
      out_specs=pl.BlockSpec((block_rows, emb), lambda i: (i, 0)),
      out_shape=jax.ShapeDtypeStruct((n, emb), jnp.float32),
  )(x, W, b2)


def _sc_combine(deg2, degree_table, feat, n_nodes):
  npad = deg2.shape[0] // NC
  vocab, emb = degree_table.shape
  rs = npad // NW
  last = n_nodes - (NW - 1) * rs
  nch = 4
  gc = rs // nch
  assert rs % (nch * L) == 0 and gc <= 128 and last == gc and last % 8 == 0

  @functools.partial(
      pl.kernel,
      out_type=jax.ShapeDtypeStruct((n_nodes, emb), jnp.float32),
      mesh=_sc_mesh(),
      compiler_params=pltpu.CompilerParams(needs_layout_passes=False),
      scratch_types=[
          pltpu.VMEM((rs,), jnp.int32),
          pltpu.VMEM((rs,), jnp.int32),
          pltpu.VMEM((rs,), jnp.int32),
          pltpu.VMEM((rs, emb), jnp.float32),
          pltpu.VMEM_SHARED((vocab, emb), jnp.float32),
          pltpu.SemaphoreType.DMA,
          pltpu.SemaphoreType.DMA,
          pltpu.SemaphoreType.DMA,
          pltpu.SemaphoreType.DMA,
          pltpu.SemaphoreType.DMA,
          pltpu.SemaphoreType.DMA,
      ],
  )
  def combine_kernel(deg_hbm, tab_hbm, feat_hbm, out_hbm,
                     da_v, db_v, ii_v, acc_v, tab_sp, sem_d, sem_o,
                     sc0, sc1, sc2, sc3):
    scs = [sc0, sc1, sc2, sc3]
    cid = lax.axis_index("c")
    sid = lax.axis_index("s")
    wid = cid * NS + sid
    base = wid * rs
    is_last = wid == NW - 1

    cpa = pltpu.async_copy(deg_hbm.at[pl.ds(base, rs)], da_v, sem_d)
    cpb = pltpu.async_copy(deg_hbm.at[pl.ds(npad + base, rs)], db_v, sem_d)

    cp0 = pltpu.async_copy(
        feat_hbm.at[pl.ds(base, gc)], acc_v.at[pl.ds(0, gc)], scs[0])

    @pl.when(~is_last)
    def _():
      for k in range(1, nch):
        pltpu.async_copy(feat_hbm.at[pl.ds(base + k * gc, gc)],
                         acc_v.at[pl.ds(k * gc, gc)], scs[k])

    tslice = vocab // NS
    pltpu.sync_copy(tab_hbm.at[pl.ds(sid * tslice, tslice)],
                    tab_sp.at[pl.ds(sid * tslice, tslice)])

    cpa.wait()
    cpb.wait()

    def ibody(j, _):
      s = da_v[pl.ds(j * L, L)] + db_v[pl.ds(j * L, L)]
      ii_v[pl.ds(j * L, L)] = jnp.minimum(s, vocab - 1)
      return 0

    lax.fori_loop(0, rs // L, ibody, 0)

    plsc.subcore_barrier()

    def fire_gather(k):
      return pltpu.async_copy(tab_sp.at[ii_v.at[pl.ds(k * gc, gc)]],
                              acc_v.at[pl.ds(k * gc, gc)], scs[k], add=True)

    def fire_out(k):
      pltpu.async_copy(acc_v.at[pl.ds(k * gc, gc)],
                       out_hbm.at[pl.ds(base + k * gc, gc)], sem_o)

    cp0.wait()
    g0 = fire_gather(0)

    @pl.when(~is_last)
    def _():
      gprev = g0
      for k in range(1, nch):
        pltpu.make_async_copy(feat_hbm.at[pl.ds(base + k * gc, gc)],
                              acc_v.at[pl.ds(k * gc, gc)], scs[k]).wait()
        gk = fire_gather(k)
        gprev.wait()
        fire_out(k - 1)
        gprev = gk
      gprev.wait()
      fire_out(nch - 1)
      for k in range(nch):
        pltpu.make_async_copy(acc_v.at[pl.ds(k * gc, gc)],
                              out_hbm.at[pl.ds(base + k * gc, gc)],
                              sem_o).wait()

    @pl.when(is_last)
    def _():
      g0.wait()
      fire_out(0)
      pltpu.make_async_copy(acc_v.at[pl.ds(0, gc)],
                            out_hbm.at[pl.ds(base, gc)], sem_o).wait()

  return combine_kernel(deg2, degree_table, feat)


def kernel(x, edge_index, W, b, degree_table):
  n = x.shape[0]
  e_total = edge_index.shape[1]
  hist_kernel, _ = _sc_degree_histogram(e_total, n)
  deg2 = hist_kernel(edge_index)
  feat = _tc_matmul(x, W, b.reshape(1, -1), block_rows=2000)
  return _sc_combine(deg2, degree_table, feat, n)


# speedup vs baseline: 1.0937x; 1.0033x over previous
import functools

import jax
import jax.numpy as jnp
from jax import lax
from jax.experimental import pallas as pl
from jax.experimental.pallas import tpu as pltpu
from jax.experimental.pallas import tpu_sc as plsc


NC = 2
NS = 16
L = 16
NW = NC * NS


def _sc_mesh():
  return plsc.VectorSubcoreMesh(core_axis_name="c", subcore_axis_name="s")


def _sc_degree_histogram(e_total, n_nodes):
  ec = 10240
  ec_last = e_total - (NW - 1) * ec
  assert ec_last > 0 and ec_last % 512 == 0
  npad = ((n_nodes + NS * L - 1) // (NS * L)) * (NS * L)
  rs = npad // NS

  @functools.partial(
      pl.kernel,
      out_type=jax.ShapeDtypeStruct((NW * npad,), jnp.int32),
      mesh=_sc_mesh(),
      compiler_params=pltpu.CompilerParams(needs_layout_passes=False),
      scratch_types=[
          pltpu.VMEM((2, ec), jnp.int32),
          pltpu.VMEM((npad,), jnp.int32),
          pltpu.SemaphoreType.DMA,
      ],
  )
  def hist_kernel(ei_hbm, deg_hbm, idx_v, hist_v, sem):
    cid = lax.axis_index("c")
    sid = lax.axis_index("s")
    wid = cid * NS + sid

    zeros = jnp.zeros((L,), jnp.int32)
    ones = jnp.ones((L,), jnp.int32)
    hu = 16

    def do_hist(csz):
      half = csz // 2
      assert half % (L * hu) == 0
      off = wid * ec
      cp0 = pltpu.async_copy(
          ei_hbm.at[:, pl.ds(off, half)], idx_v.at[:, pl.ds(0, half)], sem)
      cp1 = pltpu.async_copy(
          ei_hbm.at[:, pl.ds(off + half, half)],
          idx_v.at[:, pl.ds(half, half)], sem)

      zu = 8
      assert npad % (L * zu) == 0

      def zbody(i, _):
        for u in range(zu):
          hist_v[pl.ds((i * zu + u) * L, L)] = zeros
        return 0

      lax.fori_loop(0, npad // (L * zu), zbody, 0)

      def hbody(e, _):
        idxs = [idx_v[1, pl.ds((e * hu + u) * L, L)] for u in range(hu)]
        for u in range(hu):
          plsc.addupdate_scatter(hist_v, [idxs[u]], ones)
        return 0

      cp0.wait()
      lax.fori_loop(0, half // (L * hu), hbody, 0)
      cp1.wait()
      lax.fori_loop(half // (L * hu), csz // (L * hu), hbody, 0)

    @pl.when(wid < NW - 1)
    def _():
      do_hist(ec)

    @pl.when(wid == NW - 1)
    def _():
      do_hist(ec_last)

    pltpu.sync_copy(hist_v, deg_hbm.at[pl.ds(wid * npad, npad)])

  return hist_kernel, npad


def _tc_matmul(x, W, b2, block_rows):
  n, dfeat = x.shape
  emb = W.shape[0]
  assert n % block_rows == 0

  def body(x_ref, w_ref, b_ref, o_ref):
    o_ref[...] = lax.dot_general(
        x_ref[...], w_ref[...],
        dimension_numbers=(((1,), (1,)), ((), ())),
        preferred_element_type=jnp.float32) + b_ref[...]

  return pl.pallas_call(
      body,
      grid=(n // block_rows,),
      in_specs=[
          pl.BlockSpec((block_rows, dfeat), lambda i: (i, 0)),
          pl.BlockSpec((emb, dfeat), lambda i: (0, 0)),
          pl.BlockSpec((1, emb), lambda i: (0, 0)),
      ],
      out_specs=pl.BlockSpec((block_rows, emb), lambda i: (i, 0)),
      out_shape=jax.ShapeDtypeStruct((n, emb), jnp.float32),
  )(x, W, b2)


def _sc_combine(deg2, degree_table, feat, n_nodes):
  npad = deg2.shape[0] // NW
  vocab, emb = degree_table.shape
  rs = npad // NW
  last = n_nodes - (NW - 1) * rs
  nch = 4
  gc = rs // nch
  assert rs % (nch * L) == 0 and gc <= 128 and last == gc and last % 8 == 0

  @functools.partial(
      pl.kernel,
      out_type=jax.ShapeDtypeStruct((n_nodes, emb), jnp.float32),
      mesh=_sc_mesh(),
      compiler_params=pltpu.CompilerParams(needs_layout_passes=False),
      scratch_types=[
          pltpu.VMEM((NW * rs,), jnp.int32),
          pltpu.VMEM((rs,), jnp.int32),
          pltpu.VMEM((rs, emb), jnp.float32),
          pltpu.VMEM_SHARED((vocab, emb), jnp.float32),
          pltpu.SemaphoreType.DMA,
          pltpu.SemaphoreType.DMA,
          pltpu.SemaphoreType.DMA,
          pltpu.SemaphoreType.DMA,
          pltpu.SemaphoreType.DMA,
          pltpu.SemaphoreType.DMA,
      ],
  )
  def combine_kernel(deg_hbm, tab_hbm, feat_hbm, out_hbm,
                     pc_v, ii_v, acc_v, tab_sp, sem_d, sem_o,
                     sc0, sc1, sc2, sc3):
    scs = [sc0, sc1, sc2, sc3]
    cid = lax.axis_index("c")
    sid = lax.axis_index("s")
    wid = cid * NS + sid
    base = wid * rs
    is_last = wid == NW - 1

    pc_cp = []
    for w in range(NW):
      pc_cp.append(pltpu.async_copy(
          deg_hbm.at[pl.ds(w * npad + base, rs)],
          pc_v.at[pl.ds(w * rs, rs)], sem_d))

    cp0 = pltpu.async_copy(
        feat_hbm.at[pl.ds(base, gc)], acc_v.at[pl.ds(0, gc)], scs[0])

    @pl.when(~is_last)
    def _():
      for k in range(1, nch):
        pltpu.async_copy(feat_hbm.at[pl.ds(base + k * gc, gc)],
                         acc_v.at[pl.ds(k * gc, gc)], scs[k])

    tslice = vocab // NS
    pltpu.sync_copy(tab_hbm.at[pl.ds(sid * tslice, tslice)],
                    tab_sp.at[pl.ds(sid * tslice, tslice)])

    for cp in pc_cp:
      cp.wait()

    def ibody(j, _):
      vals = [pc_v[pl.ds(w * rs + j * L, L)] for w in range(NW)]
      while len(vals) > 1:
        vals = [a + b for a, b in zip(vals[::2], vals[1::2])]
      ii_v[pl.ds(j * L, L)] = jnp.minimum(vals[0], vocab - 1)
      return 0

    lax.fori_loop(0, rs // L, ibody, 0)

    plsc.subcore_barrier()

    def fire_gather(k):
      return pltpu.async_copy(tab_sp.at[ii_v.at[pl.ds(k * gc, gc)]],
                              acc_v.at[pl.ds(k * gc, gc)], scs[k], add=True)

    def fire_out(k):
      pltpu.async_copy(acc_v.at[pl.ds(k * gc, gc)],
                       out_hbm.at[pl.ds(base + k * gc, gc)], sem_o)

    cp0.wait()
    g0 = fire_gather(0)

    @pl.when(~is_last)
    def _():
      gprev = g0
      for k in range(1, nch):
        pltpu.make_async_copy(feat_hbm.at[pl.ds(base + k * gc, gc)],
                              acc_v.at[pl.ds(k * gc, gc)], scs[k]).wait()
        gk = fire_gather(k)
        gprev.wait()
        fire_out(k - 1)
        gprev = gk
      gprev.wait()
      fire_out(nch - 1)
      for k in range(nch):
        pltpu.make_async_copy(acc_v.at[pl.ds(k * gc, gc)],
                              out_hbm.at[pl.ds(base + k * gc, gc)],
                              sem_o).wait()

    @pl.when(is_last)
    def _():
      g0.wait()
      fire_out(0)
      pltpu.make_async_copy(acc_v.at[pl.ds(0, gc)],
                            out_hbm.at[pl.ds(base, gc)], sem_o).wait()

  return combine_kernel(deg2, degree_table, feat)


def kernel(x, edge_index, W, b, degree_table):
  n = x.shape[0]
  e_total = edge_index.shape[1]
  hist_kernel, _ = _sc_degree_histogram(e_total, n)
  deg2 = hist_kernel(edge_index)
  feat = _tc_matmul(x, W, b.reshape(1, -1), block_rows=2000)
  return _sc_combine(deg2, degree_table, feat, n)


# speedup vs baseline: 1.0971x; 1.0031x over previous
import functools

import jax
import jax.numpy as jnp
from jax import lax
from jax.experimental import pallas as pl
from jax.experimental.pallas import tpu as pltpu
from jax.experimental.pallas import tpu_sc as plsc


NC = 2
NS = 16
L = 16
NW = NC * NS


def _sc_mesh():
  return plsc.VectorSubcoreMesh(core_axis_name="c", subcore_axis_name="s")


def _sc_degree_histogram(e_total, n_nodes):
  ec = 10240
  ec_last = e_total - (NW - 1) * ec
  assert ec_last > 0 and ec_last % 512 == 0
  npad = ((n_nodes + NS * L - 1) // (NS * L)) * (NS * L)
  rs = npad // NS

  @functools.partial(
      pl.kernel,
      out_type=jax.ShapeDtypeStruct((NW * npad,), jnp.int32),
      mesh=_sc_mesh(),
      compiler_params=pltpu.CompilerParams(needs_layout_passes=False),
      scratch_types=[
          pltpu.VMEM((2, ec), jnp.int32),
          pltpu.VMEM((npad,), jnp.int32),
          pltpu.SemaphoreType.DMA,
      ],
  )
  def hist_kernel(ei_hbm, deg_hbm, idx_v, hist_v, sem):
    cid = lax.axis_index("c")
    sid = lax.axis_index("s")
    wid = cid * NS + sid

    zeros = jnp.zeros((L,), jnp.int32)
    ones = jnp.ones((L,), jnp.int32)
    hu = 16

    def do_hist(csz):
      half = csz // 2
      assert half % (L * hu) == 0
      off = wid * ec
      cp0 = pltpu.async_copy(
          ei_hbm.at[:, pl.ds(off, half)], idx_v.at[:, pl.ds(0, half)], sem)
      cp1 = pltpu.async_copy(
          ei_hbm.at[:, pl.ds(off + half, half)],
          idx_v.at[:, pl.ds(half, half)], sem)

      zu = 8
      assert npad % (L * zu) == 0

      def zbody(i, _):
        for u in range(zu):
          hist_v[pl.ds((i * zu + u) * L, L)] = zeros
        return 0

      lax.fori_loop(0, npad // (L * zu), zbody, 0)

      def hbody(e, _):
        idxs = [idx_v[1, pl.ds((e * hu + u) * L, L)] for u in range(hu)]
        for u in range(hu):
          plsc.addupdate_scatter(hist_v, [idxs[u]], ones)
        return 0

      cp0.wait()
      lax.fori_loop(0, half // (L * hu), hbody, 0)
      cp1.wait()
      lax.fori_loop(half // (L * hu), csz // (L * hu), hbody, 0)

    @pl.when(wid < NW - 1)
    def _():
      do_hist(ec)

    @pl.when(wid == NW - 1)
    def _():
      do_hist(ec_last)

    pltpu.sync_copy(hist_v, deg_hbm.at[pl.ds(wid * npad, npad)])

  return hist_kernel, npad


def _tc_matmul(x, W, b2, block_rows):
  n, dfeat = x.shape
  emb = W.shape[0]
  assert n % block_rows == 0

  def body(x_ref, w_ref, b_ref, o_ref):
    o_ref[...] = lax.dot_general(
        x_ref[...], w_ref[...],
        dimension_numbers=(((1,), (1,)), ((), ())),
        preferred_element_type=jnp.float32) + b_ref[...]

  return pl.pallas_call(
      body,
      grid=(n // block_rows,),
      in_specs=[
          pl.BlockSpec((block_rows, dfeat), lambda i: (i, 0)),
          pl.BlockSpec((emb, dfeat), lambda i: (0, 0)),
          pl.BlockSpec((1, emb), lambda i: (0, 0)),
      ],
      out_specs=pl.BlockSpec((block_rows, emb), lambda i: (i, 0)),
      out_shape=jax.ShapeDtypeStruct((n, emb), jnp.float32),
  )(x, W, b2)


def _sc_combine(deg2, degree_table, feat, n_nodes):
  npad = deg2.shape[0] // NW
  vocab, emb = degree_table.shape
  rs = npad // NW
  last = n_nodes - (NW - 1) * rs
  nch = 8
  gc = rs // nch
  lch = last // gc
  assert rs % nch == 0 and rs % L == 0 and gc <= 128 and gc % 8 == 0
  assert last % gc == 0 and 0 < lch < nch

  @functools.partial(
      pl.kernel,
      out_type=jax.ShapeDtypeStruct((n_nodes, emb), jnp.float32),
      mesh=_sc_mesh(),
      compiler_params=pltpu.CompilerParams(needs_layout_passes=False),
      scratch_types=[
          pltpu.VMEM((NW * rs,), jnp.int32),
          pltpu.VMEM((rs,), jnp.int32),
          pltpu.VMEM((rs, emb), jnp.float32),
          pltpu.VMEM_SHARED((vocab, emb), jnp.float32),
          pltpu.SemaphoreType.DMA,
          pltpu.SemaphoreType.DMA,
      ] + [pltpu.SemaphoreType.DMA] * 8,
  )
  def combine_kernel(deg_hbm, tab_hbm, feat_hbm, out_hbm,
                     pc_v, ii_v, acc_v, tab_sp, sem_d, sem_o, *scs):
    cid = lax.axis_index("c")
    sid = lax.axis_index("s")
    wid = cid * NS + sid
    base = wid * rs
    is_last = wid == NW - 1

    pc_cp = []
    for w in range(NW):
      pc_cp.append(pltpu.async_copy(
          deg_hbm.at[pl.ds(w * npad + base, rs)],
          pc_v.at[pl.ds(w * rs, rs)], sem_d))

    def fire_feat(k):
      return pltpu.async_copy(feat_hbm.at[pl.ds(base + k * gc, gc)],
                              acc_v.at[pl.ds(k * gc, gc)], scs[k])

    cps = [fire_feat(k) for k in range(lch)]

    @pl.when(~is_last)
    def _():
      for k in range(lch, nch):
        fire_feat(k)

    tslice = vocab // NS
    pltpu.sync_copy(tab_hbm.at[pl.ds(sid * tslice, tslice)],
                    tab_sp.at[pl.ds(sid * tslice, tslice)])

    for cp in pc_cp:
      cp.wait()

    def ibody(j, _):
      vals = [pc_v[pl.ds(w * rs + j * L, L)] for w in range(NW)]
      while len(vals) > 1:
        vals = [a + b for a, b in zip(vals[::2], vals[1::2])]
      ii_v[pl.ds(j * L, L)] = jnp.minimum(vals[0], vocab - 1)
      return 0

    lax.fori_loop(0, rs // L, ibody, 0)

    plsc.subcore_barrier()

    def fire_gather(k):
      return pltpu.async_copy(tab_sp.at[ii_v.at[pl.ds(k * gc, gc)]],
                              acc_v.at[pl.ds(k * gc, gc)], scs[k], add=True)

    def fire_out(k):
      pltpu.async_copy(acc_v.at[pl.ds(k * gc, gc)],
                       out_hbm.at[pl.ds(base + k * gc, gc)], sem_o)

    g = [None] * nch
    cps[0].wait()
    g[0] = fire_gather(0)
    for k in range(1, lch):
      cps[k].wait()
      g[k] = fire_gather(k)
      g[k - 1].wait()
      fire_out(k - 1)

    @pl.when(~is_last)
    def _():
      gprev = g[lch - 1]
      for k in range(lch, nch):
        pltpu.make_async_copy(feat_hbm.at[pl.ds(base + k * gc, gc)],
                              acc_v.at[pl.ds(k * gc, gc)], scs[k]).wait()
        gk = fire_gather(k)
        gprev.wait()
        fire_out(k - 1)
        gprev = gk
      gprev.wait()
      fire_out(nch - 1)
      for k in range(nch):
        pltpu.make_async_copy(acc_v.at[pl.ds(k * gc, gc)],
                              out_hbm.at[pl.ds(base + k * gc, gc)],
                              sem_o).wait()

    @pl.when(is_last)
    def _():
      g[lch - 1].wait()
      fire_out(lch - 1)
      for k in range(lch):
        pltpu.make_async_copy(acc_v.at[pl.ds(k * gc, gc)],
                              out_hbm.at[pl.ds(base + k * gc, gc)],
                              sem_o).wait()

  return combine_kernel(deg2, degree_table, feat)


def kernel(x, edge_index, W, b, degree_table):
  n = x.shape[0]
  e_total = edge_index.shape[1]
  hist_kernel, _ = _sc_degree_histogram(e_total, n)
  deg2 = hist_kernel(edge_index)
  feat = _tc_matmul(x, W, b.reshape(1, -1), block_rows=2000)
  return _sc_combine(deg2, degree_table, feat, n)
